# baseline (speedup 1.0000x reference)
import functools

import jax
import jax.numpy as jnp
from jax import lax
from jax.experimental import pallas as pl
from jax.experimental.pallas import tpu as pltpu
from jax.experimental.pallas import tpu_sc as plsc

BATCH = 4096
SEQ_LEN = 50
NUM_RESIDUES = 1000
EMBED_DIM = 64

NUM_WORKERS = 32
ROWS_PER_W = BATCH // NUM_WORKERS
NBUF = 8


def _sc_gather(indices, table):
    mesh = plsc.VectorSubcoreMesh(core_axis_name="c", subcore_axis_name="s")

    @functools.partial(
        pl.kernel,
        mesh=mesh,
        compiler_params=pltpu.CompilerParams(use_tc_tiling_on_sc=False),
        out_type=jax.ShapeDtypeStruct((BATCH, SEQ_LEN, EMBED_DIM), jnp.float32),
        scratch_types=[
            pltpu.VMEM((ROWS_PER_W, SEQ_LEN), jnp.int32),
            pltpu.VMEM((NBUF, SEQ_LEN, EMBED_DIM), jnp.float32),
        ]
        + [pltpu.SemaphoreType.DMA] * (2 * NBUF),
    )
    def k(idx_hbm, table_hbm, out_hbm, idx_v, rows_v, *sems):
        gsem, osem = sems[:NBUF], sems[NBUF:]
        wid = lax.axis_index("s") * 2 + lax.axis_index("c")
        row0 = wid * ROWS_PER_W
        pltpu.sync_copy(idx_hbm.at[pl.ds(row0, ROWS_PER_W)], idx_v)

        def fire_gather(r, b):
            pltpu.async_copy(
                table_hbm.at[idx_v.at[r]],
                rows_v.at[b],
                gsem[b],
            )

        def wait_gather(b):
            pltpu.make_async_copy(
                out_hbm.at[0], rows_v.at[b], gsem[b]
            ).wait()

        def fire_scatter(r, b):
            pltpu.async_copy(
                rows_v.at[b],
                out_hbm.at[row0 + r],
                osem[b],
            )

        def wait_scatter(b):
            pltpu.make_async_copy(
                rows_v.at[b],
                out_hbm.at[0],
                osem[b],
            ).wait()

        for b in range(NBUF):
            fire_gather(b, b)

        def body(i, carry):
            r0 = i * NBUF
            for b in range(NBUF):
                wait_gather(b)
                fire_scatter(r0 + b, b)
            for b in range(NBUF):
                wait_scatter(b)
                fire_gather(r0 + NBUF + b, b)
            return carry

        lax.fori_loop(0, ROWS_PER_W // NBUF - 1, body, 0)

        for b in range(NBUF):
            r = ROWS_PER_W - NBUF + b
            wait_gather(b)
            fire_scatter(r, b)
        for b in range(NBUF):
            wait_scatter(b)

    return k(indices, table)


def kernel(indices, embeddings):
    return _sc_gather(indices, embeddings)


# speedup vs baseline: 1.0089x; 1.0089x over previous
import functools

import jax
import jax.numpy as jnp
from jax import lax
from jax.experimental import pallas as pl
from jax.experimental.pallas import tpu as pltpu
from jax.experimental.pallas import tpu_sc as plsc

BATCH = 4096
SEQ_LEN = 50
NUM_RESIDUES = 1000
EMBED_DIM = 64

NUM_WORKERS = 32
TOTAL = BATCH * SEQ_LEN
PER_W = TOTAL // NUM_WORKERS
NBUF = 4
GROUPS = 16
GR = PER_W // GROUPS


def _sc_gather(idx_flat, table):
    mesh = plsc.VectorSubcoreMesh(core_axis_name="c", subcore_axis_name="s")

    @functools.partial(
        pl.kernel,
        mesh=mesh,
        compiler_params=pltpu.CompilerParams(use_tc_tiling_on_sc=False),
        out_type=jax.ShapeDtypeStruct((TOTAL, EMBED_DIM), jnp.float32),
        scratch_types=[
            pltpu.VMEM((PER_W,), jnp.int32),
            pltpu.VMEM((NBUF, GR, EMBED_DIM), jnp.float32),
        ]
        + [pltpu.SemaphoreType.DMA] * (2 * NBUF),
    )
    def k(idx_hbm, table_hbm, out_hbm, idx_v, rows_v, *sems):
        gsem, osem = sems[:NBUF], sems[NBUF:]
        wid = lax.axis_index("s") * 2 + lax.axis_index("c")
        base = wid * PER_W
        pltpu.sync_copy(idx_hbm.at[pl.ds(base, PER_W)], idx_v)

        def fire_gather(g, b):
            pltpu.async_copy(
                table_hbm.at[idx_v.at[pl.ds(g * GR, GR)]], rows_v.at[b], gsem[b]
            )

        def wait_gather(b):
            pltpu.make_async_copy(
                table_hbm.at[pl.ds(0, GR)], rows_v.at[b], gsem[b]
            ).wait()

        def fire_scatter(g, b):
            pltpu.async_copy(
                rows_v.at[b],
                out_hbm.at[pl.ds(base + g * GR, GR)],
                osem[b],
            )

        def wait_scatter(b):
            pltpu.make_async_copy(
                rows_v.at[b],
                out_hbm.at[pl.ds(0, GR)],
                osem[b],
            ).wait()

        for b in range(NBUF):
            fire_gather(b, b)

        def body(i, carry):
            g0 = i * NBUF
            for b in range(NBUF):
                wait_gather(b)
                fire_scatter(g0 + b, b)
            for b in range(NBUF):
                wait_scatter(b)
                fire_gather(g0 + NBUF + b, b)
            return carry

        lax.fori_loop(0, GROUPS // NBUF - 1, body, 0)

        for b in range(NBUF):
            g = GROUPS - NBUF + b
            wait_gather(b)
            fire_scatter(g, b)
        for b in range(NBUF):
            wait_scatter(b)

    return k(idx_flat, table)


def kernel(indices, embeddings):
    idx = jnp.clip(indices, 0, NUM_RESIDUES - 1)
    out = _sc_gather(idx.reshape(TOTAL), embeddings)
    return out.reshape(BATCH, SEQ_LEN, EMBED_DIM)
